# baseline (speedup 1.0000x reference)
import jax
import jax.numpy as jnp
from jax import lax
from jax.experimental import pallas as pl
from jax.experimental.pallas import tpu as pltpu
from jax.experimental.pallas import tpu_sc as plsc

T = 4096
HIDDEN = 1024
E = 8
TOPK = 2
DFF = 512
NGROUP = 4
TOPK_GROUP = 2
ROUTED_SCALING = 2.5

NPAIR = TOPK * T
BTR = 1024
PBUF = NPAIR + E * BTR
NBLK = PBUF // BTR

SC_CORES = 2
SC_SUBCORES = 16
SC_WORKERS = SC_CORES * SC_SUBCORES
SC_CHUNK = 32
SC_CHUNKS_PER_WORKER = NPAIR // (SC_CHUNK * SC_WORKERS)

BTS = 256


def _lane_cumsum(x, n):
    s = 1
    while s < n:
        shifted = jnp.concatenate([jnp.zeros_like(x[:, :s]), x[:, :-s]],
                                  axis=1)
        x = x + shifted
        s *= 2
    return x


def _route_body(hid_ref, gw_ref, bias_ref, wtok_ref, dest_ref, bexp_ref):
    h = hid_ref[...]

    logits = lax.dot_general(gw_ref[...], h, (((0,), (1,)), ((), ())),
                             preferred_element_type=jnp.float32)
    s = jax.nn.sigmoid(logits)
    sc = s + bias_ref[...]

    srows = [sc[e:e + 1] for e in range(E)]
    grows = [srows[2 * g] + srows[2 * g + 1] for g in range(NGROUP)]

    keep_group = []
    for g in range(NGROUP):
        rank = jnp.zeros((1, T), jnp.int32)
        for g2 in range(NGROUP):
            if g2 == g:
                continue
            beats = (grows[g2] >= grows[g]) if g2 < g else (grows[g2] > grows[g])
            rank += beats.astype(jnp.int32)
        keep_group.append(rank < TOPK_GROUP)

    neg = jnp.float32(-jnp.inf)
    mrows = [jnp.where(keep_group[e // 2], srows[e], neg) for e in range(E)]
    keep = []
    for e in range(E):
        rank = jnp.zeros((1, T), jnp.int32)
        for e2 in range(E):
            if e2 == e:
                continue
            beats = (mrows[e2] >= mrows[e]) if e2 < e else (mrows[e2] > mrows[e])
            rank += beats.astype(jnp.int32)
        keep.append(rank < TOPK)

    wrows = [jnp.where(keep[e], s[e:e + 1], 0.0) for e in range(E)]
    wsum = wrows[0]
    for e in range(1, E):
        wsum = wsum + wrows[e]
    wn = [w / wsum * ROUTED_SCALING for w in wrows]

    e0 = jnp.full((1, T), E, jnp.int32)
    e1 = jnp.full((1, T), -1, jnp.int32)
    for e in range(E):
        ei = jnp.full((1, T), e, jnp.int32)
        e0 = jnp.where(keep[e], jnp.minimum(e0, ei), e0)
        e1 = jnp.where(keep[e], jnp.maximum(e1, ei), e1)

    w0 = jnp.zeros((1, T), jnp.float32)
    w1 = jnp.zeros((1, T), jnp.float32)
    for e in range(E):
        w0 = jnp.where(e0 == e, wn[e], w0)
        w1 = jnp.where(e1 == e, wn[e], w1)

    excl = []
    cnt = []
    for e in range(E):
        k = keep[e].astype(jnp.int32)
        c = _lane_cumsum(k, T)
        excl.append(c - k)
        cnt.append(c[:, T - 1:T])

    pstart = []
    run = jnp.zeros((1, 1), jnp.int32)
    for e in range(E):
        pstart.append(run)
        padded = ((cnt[e] + (BTR - 1)) // BTR) * BTR
        run = run + padded

    dest0 = jnp.zeros((1, T), jnp.int32)
    dest1 = jnp.zeros((1, T), jnp.int32)
    for e in range(E):
        d_e = pstart[e] + excl[e]
        dest0 = jnp.where(e0 == e, d_e, dest0)
        dest1 = jnp.where(e1 == e, d_e, dest1)
    dest_ref[0:1, :] = dest0
    dest_ref[1:2, :] = dest1

    bstart = jax.lax.broadcasted_iota(jnp.int32, (1, NBLK + 1), 1) * BTR
    be = jnp.full((1, NBLK + 1), -1, jnp.int32)
    for e in range(E):
        be = be + (pstart[e] <= bstart).astype(jnp.int32)
    nreal = run // BTR
    lane = jax.lax.broadcasted_iota(jnp.int32, (1, NBLK + 1), 1)
    bexp_ref[...] = jnp.where(lane == NBLK, nreal, be)

    w01 = jnp.concatenate([w0, w1] + [jnp.zeros((1, T), jnp.float32)] * (E - 2),
                          axis=0)
    eye = jnp.eye(E, dtype=jnp.float32)
    wtok_ref[...] = lax.dot_general(w01, eye, (((0,), (0,)), ((), ())),
                                    preferred_element_type=jnp.float32)


@jax.jit
def _route(hidden_states, gate_w, bias_col):
    return pl.pallas_call(
        _route_body,
        grid=(1,),
        in_specs=[
            pl.BlockSpec((T, HIDDEN), lambda i: (0, 0)),
            pl.BlockSpec((HIDDEN, E), lambda i: (0, 0)),
            pl.BlockSpec((E, 1), lambda i: (0, 0)),
        ],
        out_specs=[
            pl.BlockSpec((T, E), lambda i: (0, 0)),
            pl.BlockSpec((2, T), lambda i: (0, 0)),
            pl.BlockSpec((1, NBLK + 1), lambda i: (0, 0)),
        ],
        out_shape=[
            jax.ShapeDtypeStruct((T, E), jnp.float32),
            jax.ShapeDtypeStruct((2, T), jnp.int32),
            jax.ShapeDtypeStruct((1, NBLK + 1), jnp.int32),
        ],
    )(hidden_states, gate_w, bias_col)


def _dispatch_body(hid_ref, dest_ref, sorted_ref, idx0, idx1, rows0, rows1,
                   rsem0, rsem1, wsem0, wsem1):
    wid = lax.axis_index("s") * SC_CORES + lax.axis_index("c")
    idx = (idx0, idx1)
    rows = (rows0, rows1)
    rsem = (rsem0, rsem1)
    wsem = (wsem0, wsem1)
    n = SC_CHUNKS_PER_WORKER

    def chunk_base(cc):
        base = (wid * n + cc) * SC_CHUNK
        return base, lax.rem(base, T)

    b0, t0 = chunk_base(0)
    pltpu.sync_copy(dest_ref.at[pl.ds(b0, SC_CHUNK)], idx[0])
    reads = [None, None]
    writes = [None, None]
    reads[0] = pltpu.async_copy(hid_ref.at[pl.ds(t0, SC_CHUNK)], rows[0],
                                rsem[0])
    for cc in range(n):
        cur = cc % 2
        nxt = 1 - cur
        if cc + 1 < n:
            if writes[nxt] is not None:
                writes[nxt].wait()
            bn, tn = chunk_base(cc + 1)
            pltpu.sync_copy(dest_ref.at[pl.ds(bn, SC_CHUNK)], idx[nxt])
            reads[nxt] = pltpu.async_copy(hid_ref.at[pl.ds(tn, SC_CHUNK)],
                                          rows[nxt], rsem[nxt])
        reads[cur].wait()
        writes[cur] = pltpu.async_copy(rows[cur], sorted_ref.at[idx[cur]],
                                       wsem[cur])
    for w in writes:
        if w is not None:
            w.wait()


@jax.jit
def _dispatch(hidden_states, dest_flat):
    mesh = plsc.VectorSubcoreMesh(core_axis_name="c", subcore_axis_name="s")
    fn = pl.kernel(
        _dispatch_body,
        out_type=jax.ShapeDtypeStruct((PBUF, HIDDEN), jnp.float32),
        mesh=mesh,
        scratch_types=[
            pltpu.VMEM((SC_CHUNK,), jnp.int32),
            pltpu.VMEM((SC_CHUNK,), jnp.int32),
            pltpu.VMEM((SC_CHUNK, HIDDEN), jnp.float32),
            pltpu.VMEM((SC_CHUNK, HIDDEN), jnp.float32),
            pltpu.SemaphoreType.DMA,
            pltpu.SemaphoreType.DMA,
            pltpu.SemaphoreType.DMA,
            pltpu.SemaphoreType.DMA,
        ],
    )
    return fn(hidden_states, dest_flat)


def _expert_body(bexp_ref, sorted_ref, wgu_ref, wd_ref, out_ref):
    @pl.when(pl.program_id(0) < bexp_ref[NBLK])
    def _():
        x = sorted_ref[...].astype(jnp.bfloat16)
        gu = jnp.dot(x, wgu_ref[0].astype(jnp.bfloat16),
                     preferred_element_type=jnp.float32)
        g = gu[:, :DFF]
        u = gu[:, DFF:]
        act = (g * jax.nn.sigmoid(g) * u).astype(jnp.bfloat16)
        out_ref[...] = jnp.dot(act, wd_ref[0].astype(jnp.bfloat16),
                               preferred_element_type=jnp.float32)


@jax.jit
def _expert_mlp(block_expert, sorted_h, w_gate_up, w_down):
    grid_spec = pltpu.PrefetchScalarGridSpec(
        num_scalar_prefetch=1,
        grid=(NBLK,),
        in_specs=[
            pl.BlockSpec((BTR, HIDDEN), lambda i, be: (i, 0)),
            pl.BlockSpec((1, HIDDEN, 2 * DFF), lambda i, be: (be[i], 0, 0)),
            pl.BlockSpec((1, DFF, HIDDEN), lambda i, be: (be[i], 0, 0)),
        ],
        out_specs=pl.BlockSpec((BTR, HIDDEN), lambda i, be: (i, 0)),
    )
    return pl.pallas_call(
        _expert_body,
        grid_spec=grid_spec,
        out_shape=jax.ShapeDtypeStruct((PBUF, HIDDEN), jnp.float32),
    )(block_expert, sorted_h, w_gate_up, w_down)


def _unpermute_body(rowout_ref, dest_ref, unsorted_ref, idx0, idx1, rows0,
                    rows1, rsem0, rsem1, wsem0, wsem1):
    wid = lax.axis_index("s") * SC_CORES + lax.axis_index("c")
    idx = (idx0, idx1)
    rows = (rows0, rows1)
    rsem = (rsem0, rsem1)
    wsem = (wsem0, wsem1)
    n = SC_CHUNKS_PER_WORKER

    b0 = wid * n * SC_CHUNK
    pltpu.sync_copy(dest_ref.at[pl.ds(b0, SC_CHUNK)], idx[0])
    reads = [None, None]
    writes = [None, None]
    reads[0] = pltpu.async_copy(rowout_ref.at[idx[0]], rows[0], rsem[0])
    for cc in range(n):
        cur = cc % 2
        nxt = 1 - cur
        base = (wid * n + cc) * SC_CHUNK
        if cc + 1 < n:
            if writes[nxt] is not None:
                writes[nxt].wait()
            bn = (wid * n + cc + 1) * SC_CHUNK
            pltpu.sync_copy(dest_ref.at[pl.ds(bn, SC_CHUNK)], idx[nxt])
            reads[nxt] = pltpu.async_copy(rowout_ref.at[idx[nxt]], rows[nxt],
                                          rsem[nxt])
        reads[cur].wait()
        writes[cur] = pltpu.async_copy(
            rows[cur], unsorted_ref.at[pl.ds(base, SC_CHUNK)], wsem[cur])
    for w in writes:
        if w is not None:
            w.wait()


@jax.jit
def _unpermute(rowout, dest_flat):
    mesh = plsc.VectorSubcoreMesh(core_axis_name="c", subcore_axis_name="s")
    fn = pl.kernel(
        _unpermute_body,
        out_type=jax.ShapeDtypeStruct((NPAIR, HIDDEN), jnp.float32),
        mesh=mesh,
        scratch_types=[
            pltpu.VMEM((SC_CHUNK,), jnp.int32),
            pltpu.VMEM((SC_CHUNK,), jnp.int32),
            pltpu.VMEM((SC_CHUNK, HIDDEN), jnp.float32),
            pltpu.VMEM((SC_CHUNK, HIDDEN), jnp.float32),
            pltpu.SemaphoreType.DMA,
            pltpu.SemaphoreType.DMA,
            pltpu.SemaphoreType.DMA,
            pltpu.SemaphoreType.DMA,
        ],
    )
    return fn(rowout, dest_flat)


def _shared_body(hid_ref, wshgu_ref, wshd_ref, out_ref):
    h = hid_ref[...].astype(jnp.bfloat16)
    gu = jnp.dot(h, wshgu_ref[...].astype(jnp.bfloat16),
                 preferred_element_type=jnp.float32)
    sg = gu[:, :2 * DFF]
    su = gu[:, 2 * DFF:]
    act = (sg * jax.nn.sigmoid(sg) * su).astype(jnp.bfloat16)
    out_ref[...] = jnp.dot(act, wshd_ref[...].astype(jnp.bfloat16),
                           preferred_element_type=jnp.float32)


def _make_shared_half(offset_blocks):
    nb = (T // 2) // BTS

    @jax.jit
    def fn(hidden_states, shared_w_gate_up, shared_w_down):
        return pl.pallas_call(
            _shared_body,
            grid=(nb,),
            in_specs=[
                pl.BlockSpec((BTS, HIDDEN),
                             lambda i, _o=offset_blocks: (i + _o, 0)),
                pl.BlockSpec((HIDDEN, 4 * DFF), lambda i: (0, 0)),
                pl.BlockSpec((2 * DFF, HIDDEN), lambda i: (0, 0)),
            ],
            out_specs=pl.BlockSpec((BTS, HIDDEN), lambda i: (i, 0)),
            out_shape=jax.ShapeDtypeStruct((T // 2, HIDDEN), jnp.float32),
        )(hidden_states, shared_w_gate_up, shared_w_down)

    return fn


_shared_mlp_a = _make_shared_half(0)
_shared_mlp_b = _make_shared_half((T // 2) // BTS)


def _combine_body(sha_ref, shb_ref, u0_ref, u1_ref, wtok_ref, out_ref):
    nba = (T // 2) // BTS
    w0 = wtok_ref[:, 0:1]
    w1 = wtok_ref[:, 1:2]
    sh = jnp.where(pl.program_id(0) < nba, sha_ref[...], shb_ref[...])
    out_ref[...] = sh + w0 * u0_ref[...] + w1 * u1_ref[...]


@jax.jit
def _combine(shared_a, shared_b, unsorted, wtok):
    nb = T // BTS
    nba = (T // 2) // BTS
    return pl.pallas_call(
        _combine_body,
        grid=(nb,),
        in_specs=[
            pl.BlockSpec((BTS, HIDDEN),
                         lambda i, _m=nba - 1: (jnp.minimum(i, _m), 0)),
            pl.BlockSpec((BTS, HIDDEN),
                         lambda i, _a=nba: (jnp.maximum(i - _a, 0), 0)),
            pl.BlockSpec((BTS, HIDDEN), lambda i: (i, 0)),
            pl.BlockSpec((BTS, HIDDEN), lambda i, _nb=nb: (i + _nb, 0)),
            pl.BlockSpec((BTS, E), lambda i: (i, 0)),
        ],
        out_specs=pl.BlockSpec((BTS, HIDDEN), lambda i: (i, 0)),
        out_shape=jax.ShapeDtypeStruct((T, HIDDEN), jnp.float32),
    )(shared_a, shared_b, unsorted, unsorted, wtok)


def kernel(hidden_states, gate_w, e_score_correction_bias, w_gate_up, w_down,
           shared_w_gate_up, shared_w_down):
    bias_col = e_score_correction_bias.reshape(E, 1)
    wtok, dest01, block_expert = _route(hidden_states, gate_w, bias_col)
    dest_flat = dest01.reshape(NPAIR)
    sorted_h = _dispatch(hidden_states, dest_flat)
    shared_a = _shared_mlp_a(hidden_states, shared_w_gate_up, shared_w_down)
    be, shared_a = lax.optimization_barrier(
        (block_expert.reshape(NBLK + 1), shared_a))
    rowout = _expert_mlp(be, sorted_h, w_gate_up, w_down)
    unsorted = _unpermute(rowout, dest_flat)
    shared_b = _shared_mlp_b(hidden_states, shared_w_gate_up, shared_w_down)
    return _combine(shared_a, shared_b, unsorted, wtok)


# speedup vs baseline: 1.0143x; 1.0143x over previous
import jax
import jax.numpy as jnp
from jax import lax
from jax.experimental import pallas as pl
from jax.experimental.pallas import tpu as pltpu
from jax.experimental.pallas import tpu_sc as plsc

T = 4096
HIDDEN = 1024
E = 8
TOPK = 2
DFF = 512
NGROUP = 4
TOPK_GROUP = 2
ROUTED_SCALING = 2.5

NPAIR = TOPK * T
BTR = 512
PBUF = NPAIR + E * BTR
NBLK = PBUF // BTR

SC_CORES = 2
SC_SUBCORES = 16
SC_WORKERS = SC_CORES * SC_SUBCORES
SC_CHUNK = 32
SC_CHUNKS_PER_WORKER = NPAIR // (SC_CHUNK * SC_WORKERS)

BTS = 256


def _lane_cumsum(x, n):
    s = 1
    while s < n:
        shifted = jnp.concatenate([jnp.zeros_like(x[:, :s]), x[:, :-s]],
                                  axis=1)
        x = x + shifted
        s *= 2
    return x


def _roll_up(x, s):
    return jnp.concatenate([x[s:], x[:s]], axis=0)


def _route_body(hid_ref, gw_ref, bias_ref, wtok_ref, dest_ref, bexp_ref):
    h = hid_ref[...]

    logits = lax.dot_general(gw_ref[...], h, (((0,), (1,)), ((), ())),
                             preferred_element_type=jnp.float32)
    s = jax.nn.sigmoid(logits)
    sc = s + bias_ref[...]

    gs = jnp.concatenate(
        [sc[2 * g:2 * g + 1] + sc[2 * g + 1:2 * g + 2] for g in range(NGROUP)],
        axis=0)

    def rank_of(x, n):
        rank = jnp.zeros_like(x, dtype=jnp.int32)
        row = jax.lax.broadcasted_iota(jnp.int32, x.shape, 0)
        for r in range(1, n):
            other = _roll_up(x, r)
            gt = jnp.where(other > x, 1, 0)
            eq = jnp.where(other == x, 1, 0)
            low = jnp.where(row >= n - r, 1, 0)
            rank = rank + gt + eq * low
        return rank

    keep_group = jnp.where(rank_of(gs, NGROUP) < TOPK_GROUP, 1, 0)
    kg_expanded = jnp.concatenate(
        [keep_group[e // 2:e // 2 + 1] for e in range(E)], axis=0)

    neg = jnp.float32(-jnp.inf)
    masked = jnp.where(kg_expanded > 0, sc, neg)
    keep = rank_of(masked, E) < TOPK

    w = jnp.where(keep, s, 0.0)
    wsum = jnp.sum(w, axis=0, keepdims=True)
    wn = w / wsum * ROUTED_SCALING

    erow = jax.lax.broadcasted_iota(jnp.int32, (E, T), 0)
    e0 = jnp.min(jnp.where(keep, erow, E), axis=0, keepdims=True)
    e1 = jnp.max(jnp.where(keep, erow, -1), axis=0, keepdims=True)
    sel0 = erow == e0
    sel1 = erow == e1
    w0 = jnp.sum(jnp.where(sel0, wn, 0.0), axis=0, keepdims=True)
    w1 = jnp.sum(jnp.where(sel1, wn, 0.0), axis=0, keepdims=True)

    k_int = jnp.where(keep, 1, 0)
    csum = _lane_cumsum(k_int, T)
    excl = csum - k_int
    cnt = csum[:, T - 1:T]

    padded = ((cnt + (BTR - 1)) // BTR) * BTR
    incl = padded
    sh = 1
    while sh < E:
        incl = incl + jnp.concatenate(
            [jnp.zeros((sh, 1), jnp.int32), incl[:-sh]], axis=0)
        sh *= 2
    pstart_i = incl - padded
    run = incl[E - 1:E]

    d_e = pstart_i + excl
    dest0 = jnp.sum(jnp.where(sel0, d_e, 0), axis=0, keepdims=True)
    dest1 = jnp.sum(jnp.where(sel1, d_e, 0), axis=0, keepdims=True)
    dest_ref[0:1, :] = dest0
    dest_ref[1:2, :] = dest1

    bstart = jax.lax.broadcasted_iota(jnp.int32, (1, NBLK + 1), 1) * BTR
    be = jnp.full((1, NBLK + 1), -1, jnp.int32)
    for e in range(E):
        be = be + (pstart_i[e:e + 1] <= bstart).astype(jnp.int32)
    nreal = run // BTR
    lane = jax.lax.broadcasted_iota(jnp.int32, (1, NBLK + 1), 1)
    bexp_ref[...] = jnp.where(lane == NBLK, nreal, be)

    w01 = jnp.concatenate([w0, w1] + [jnp.zeros((1, T), jnp.float32)] * (E - 2),
                          axis=0)
    eye = jnp.eye(E, dtype=jnp.float32)
    wtok_ref[...] = lax.dot_general(w01, eye, (((0,), (0,)), ((), ())),
                                    preferred_element_type=jnp.float32)


@jax.jit
def _route(hidden_states, gate_w, bias_col):
    return pl.pallas_call(
        _route_body,
        grid=(1,),
        in_specs=[
            pl.BlockSpec((T, HIDDEN), lambda i: (0, 0)),
            pl.BlockSpec((HIDDEN, E), lambda i: (0, 0)),
            pl.BlockSpec((E, 1), lambda i: (0, 0)),
        ],
        out_specs=[
            pl.BlockSpec((T, E), lambda i: (0, 0)),
            pl.BlockSpec((2, T), lambda i: (0, 0)),
            pl.BlockSpec((1, NBLK + 1), lambda i: (0, 0)),
        ],
        out_shape=[
            jax.ShapeDtypeStruct((T, E), jnp.float32),
            jax.ShapeDtypeStruct((2, T), jnp.int32),
            jax.ShapeDtypeStruct((1, NBLK + 1), jnp.int32),
        ],
    )(hidden_states, gate_w, bias_col)


def _dispatch_body(hid_ref, dest_ref, sorted_ref, idx0, idx1, rows0, rows1,
                   rsem0, rsem1, wsem0, wsem1):
    wid = lax.axis_index("s") * SC_CORES + lax.axis_index("c")
    idx = (idx0, idx1)
    rows = (rows0, rows1)
    rsem = (rsem0, rsem1)
    wsem = (wsem0, wsem1)
    n = SC_CHUNKS_PER_WORKER

    def chunk_base(cc):
        base = (wid * n + cc) * SC_CHUNK
        return base, lax.rem(base, T)

    b0, t0 = chunk_base(0)
    pltpu.sync_copy(dest_ref.at[pl.ds(b0, SC_CHUNK)], idx[0])
    reads = [None, None]
    writes = [None, None]
    reads[0] = pltpu.async_copy(hid_ref.at[pl.ds(t0, SC_CHUNK)], rows[0],
                                rsem[0])
    for cc in range(n):
        cur = cc % 2
        nxt = 1 - cur
        if cc + 1 < n:
            if writes[nxt] is not None:
                writes[nxt].wait()
            bn, tn = chunk_base(cc + 1)
            pltpu.sync_copy(dest_ref.at[pl.ds(bn, SC_CHUNK)], idx[nxt])
            reads[nxt] = pltpu.async_copy(hid_ref.at[pl.ds(tn, SC_CHUNK)],
                                          rows[nxt], rsem[nxt])
        reads[cur].wait()
        writes[cur] = pltpu.async_copy(rows[cur], sorted_ref.at[idx[cur]],
                                       wsem[cur])
    for w in writes:
        if w is not None:
            w.wait()


@jax.jit
def _dispatch(hidden_states, dest_flat):
    mesh = plsc.VectorSubcoreMesh(core_axis_name="c", subcore_axis_name="s")
    fn = pl.kernel(
        _dispatch_body,
        out_type=jax.ShapeDtypeStruct((PBUF, HIDDEN), jnp.float32),
        mesh=mesh,
        scratch_types=[
            pltpu.VMEM((SC_CHUNK,), jnp.int32),
            pltpu.VMEM((SC_CHUNK,), jnp.int32),
            pltpu.VMEM((SC_CHUNK, HIDDEN), jnp.float32),
            pltpu.VMEM((SC_CHUNK, HIDDEN), jnp.float32),
            pltpu.SemaphoreType.DMA,
            pltpu.SemaphoreType.DMA,
            pltpu.SemaphoreType.DMA,
            pltpu.SemaphoreType.DMA,
        ],
    )
    return fn(hidden_states, dest_flat)


def _expert_body(bexp_ref, sorted_ref, wgu_ref, wd_ref, out_ref):
    @pl.when(pl.program_id(0) < bexp_ref[NBLK])
    def _():
        x = sorted_ref[...].astype(jnp.bfloat16)
        gu = jnp.dot(x, wgu_ref[0].astype(jnp.bfloat16),
                     preferred_element_type=jnp.float32)
        g = gu[:, :DFF]
        u = gu[:, DFF:]
        act = (g * jax.nn.sigmoid(g) * u).astype(jnp.bfloat16)
        out_ref[...] = jnp.dot(act, wd_ref[0].astype(jnp.bfloat16),
                               preferred_element_type=jnp.float32)


@jax.jit
def _expert_mlp(block_expert, sorted_h, w_gate_up, w_down):
    grid_spec = pltpu.PrefetchScalarGridSpec(
        num_scalar_prefetch=1,
        grid=(NBLK,),
        in_specs=[
            pl.BlockSpec((BTR, HIDDEN), lambda i, be: (i, 0)),
            pl.BlockSpec((1, HIDDEN, 2 * DFF), lambda i, be: (be[i], 0, 0)),
            pl.BlockSpec((1, DFF, HIDDEN), lambda i, be: (be[i], 0, 0)),
        ],
        out_specs=pl.BlockSpec((BTR, HIDDEN), lambda i, be: (i, 0)),
    )
    return pl.pallas_call(
        _expert_body,
        grid_spec=grid_spec,
        out_shape=jax.ShapeDtypeStruct((PBUF, HIDDEN), jnp.float32),
    )(block_expert, sorted_h, w_gate_up, w_down)


def _unpermute_body(rowout_ref, dest_ref, unsorted_ref, idx0, idx1, rows0,
                    rows1, rsem0, rsem1, wsem0, wsem1):
    wid = lax.axis_index("s") * SC_CORES + lax.axis_index("c")
    idx = (idx0, idx1)
    rows = (rows0, rows1)
    rsem = (rsem0, rsem1)
    wsem = (wsem0, wsem1)
    n = SC_CHUNKS_PER_WORKER

    b0 = wid * n * SC_CHUNK
    pltpu.sync_copy(dest_ref.at[pl.ds(b0, SC_CHUNK)], idx[0])
    reads = [None, None]
    writes = [None, None]
    reads[0] = pltpu.async_copy(rowout_ref.at[idx[0]], rows[0], rsem[0])
    for cc in range(n):
        cur = cc % 2
        nxt = 1 - cur
        base = (wid * n + cc) * SC_CHUNK
        if cc + 1 < n:
            if writes[nxt] is not None:
                writes[nxt].wait()
            bn = (wid * n + cc + 1) * SC_CHUNK
            pltpu.sync_copy(dest_ref.at[pl.ds(bn, SC_CHUNK)], idx[nxt])
            reads[nxt] = pltpu.async_copy(rowout_ref.at[idx[nxt]], rows[nxt],
                                          rsem[nxt])
        reads[cur].wait()
        writes[cur] = pltpu.async_copy(
            rows[cur], unsorted_ref.at[pl.ds(base, SC_CHUNK)], wsem[cur])
    for w in writes:
        if w is not None:
            w.wait()


@jax.jit
def _unpermute(rowout, dest_flat):
    mesh = plsc.VectorSubcoreMesh(core_axis_name="c", subcore_axis_name="s")
    fn = pl.kernel(
        _unpermute_body,
        out_type=jax.ShapeDtypeStruct((NPAIR, HIDDEN), jnp.float32),
        mesh=mesh,
        scratch_types=[
            pltpu.VMEM((SC_CHUNK,), jnp.int32),
            pltpu.VMEM((SC_CHUNK,), jnp.int32),
            pltpu.VMEM((SC_CHUNK, HIDDEN), jnp.float32),
            pltpu.VMEM((SC_CHUNK, HIDDEN), jnp.float32),
            pltpu.SemaphoreType.DMA,
            pltpu.SemaphoreType.DMA,
            pltpu.SemaphoreType.DMA,
            pltpu.SemaphoreType.DMA,
        ],
    )
    return fn(rowout, dest_flat)


def _shared_body(hid_ref, wshgu_ref, wshd_ref, out_ref):
    h = hid_ref[...].astype(jnp.bfloat16)
    gu = jnp.dot(h, wshgu_ref[...].astype(jnp.bfloat16),
                 preferred_element_type=jnp.float32)
    sg = gu[:, :2 * DFF]
    su = gu[:, 2 * DFF:]
    act = (sg * jax.nn.sigmoid(sg) * su).astype(jnp.bfloat16)
    out_ref[...] = jnp.dot(act, wshd_ref[...].astype(jnp.bfloat16),
                           preferred_element_type=jnp.float32)


def _make_shared_half(offset_blocks):
    nb = (T // 2) // BTS

    @jax.jit
    def fn(hidden_states, shared_w_gate_up, shared_w_down):
        return pl.pallas_call(
            _shared_body,
            grid=(nb,),
            in_specs=[
                pl.BlockSpec((BTS, HIDDEN),
                             lambda i, _o=offset_blocks: (i + _o, 0)),
                pl.BlockSpec((HIDDEN, 4 * DFF), lambda i: (0, 0)),
                pl.BlockSpec((2 * DFF, HIDDEN), lambda i: (0, 0)),
            ],
            out_specs=pl.BlockSpec((BTS, HIDDEN), lambda i: (i, 0)),
            out_shape=jax.ShapeDtypeStruct((T // 2, HIDDEN), jnp.float32),
        )(hidden_states, shared_w_gate_up, shared_w_down)

    return fn


_shared_mlp_a = _make_shared_half(0)
_shared_mlp_b = _make_shared_half((T // 2) // BTS)


def _combine_body(sha_ref, shb_ref, u0_ref, u1_ref, wtok_ref, out_ref):
    nba = (T // 2) // BTS
    w0 = wtok_ref[:, 0:1]
    w1 = wtok_ref[:, 1:2]
    sh = jnp.where(pl.program_id(0) < nba, sha_ref[...], shb_ref[...])
    out_ref[...] = sh + w0 * u0_ref[...] + w1 * u1_ref[...]


@jax.jit
def _combine(shared_a, shared_b, unsorted, wtok):
    nb = T // BTS
    nba = (T // 2) // BTS
    return pl.pallas_call(
        _combine_body,
        grid=(nb,),
        in_specs=[
            pl.BlockSpec((BTS, HIDDEN),
                         lambda i, _m=nba - 1: (jnp.minimum(i, _m), 0)),
            pl.BlockSpec((BTS, HIDDEN),
                         lambda i, _a=nba: (jnp.maximum(i - _a, 0), 0)),
            pl.BlockSpec((BTS, HIDDEN), lambda i: (i, 0)),
            pl.BlockSpec((BTS, HIDDEN), lambda i, _nb=nb: (i + _nb, 0)),
            pl.BlockSpec((BTS, E), lambda i: (i, 0)),
        ],
        out_specs=pl.BlockSpec((BTS, HIDDEN), lambda i: (i, 0)),
        out_shape=jax.ShapeDtypeStruct((T, HIDDEN), jnp.float32),
    )(shared_a, shared_b, unsorted, unsorted, wtok)


def kernel(hidden_states, gate_w, e_score_correction_bias, w_gate_up, w_down,
           shared_w_gate_up, shared_w_down):
    bias_col = e_score_correction_bias.reshape(E, 1)
    wtok, dest01, block_expert = _route(hidden_states, gate_w, bias_col)
    dest_flat = dest01.reshape(NPAIR)
    sorted_h = _dispatch(hidden_states, dest_flat)
    shared_a = _shared_mlp_a(hidden_states, shared_w_gate_up, shared_w_down)
    be, shared_a = lax.optimization_barrier(
        (block_expert.reshape(NBLK + 1), shared_a))
    rowout = _expert_mlp(be, sorted_h, w_gate_up, w_down)
    unsorted = _unpermute(rowout, dest_flat)
    shared_b = _shared_mlp_b(hidden_states, shared_w_gate_up, shared_w_down)
    return _combine(shared_a, shared_b, unsorted, wtok)


# speedup vs baseline: 1.0224x; 1.0080x over previous
import jax
import jax.numpy as jnp
from jax import lax
from jax.experimental import pallas as pl
from jax.experimental.pallas import tpu as pltpu
from jax.experimental.pallas import tpu_sc as plsc

T = 4096
HIDDEN = 1024
E = 8
TOPK = 2
DFF = 512
NGROUP = 4
TOPK_GROUP = 2
ROUTED_SCALING = 2.5

NPAIR = TOPK * T
BTR = 512
PBUF = NPAIR + E * BTR
NBLK = PBUF // BTR

SC_CORES = 2
SC_SUBCORES = 16
SC_WORKERS = SC_CORES * SC_SUBCORES
SC_CHUNK = 32
SC_CHUNKS_PER_WORKER = NPAIR // (SC_CHUNK * SC_WORKERS)

BTS = 512


def _lane_cumsum(x, n):
    s = 1
    while s < n:
        shifted = jnp.concatenate([jnp.zeros_like(x[:, :s]), x[:, :-s]],
                                  axis=1)
        x = x + shifted
        s *= 2
    return x


def _roll_up(x, s):
    return jnp.concatenate([x[s:], x[:s]], axis=0)


def _route_body(hid_ref, gw_ref, bias_ref, wtok_ref, dest_ref, bexp_ref):
    h = hid_ref[...]

    logits = lax.dot_general(gw_ref[...], h, (((0,), (1,)), ((), ())),
                             preferred_element_type=jnp.float32)
    s = jax.nn.sigmoid(logits)
    sc = s + bias_ref[...]

    gs = jnp.concatenate(
        [sc[2 * g:2 * g + 1] + sc[2 * g + 1:2 * g + 2] for g in range(NGROUP)],
        axis=0)

    def rank_of(x, n):
        rank = jnp.zeros_like(x, dtype=jnp.int32)
        row = jax.lax.broadcasted_iota(jnp.int32, x.shape, 0)
        for r in range(1, n):
            other = _roll_up(x, r)
            gt = jnp.where(other > x, 1, 0)
            eq = jnp.where(other == x, 1, 0)
            low = jnp.where(row >= n - r, 1, 0)
            rank = rank + gt + eq * low
        return rank

    keep_group = jnp.where(rank_of(gs, NGROUP) < TOPK_GROUP, 1, 0)
    kg_expanded = jnp.concatenate(
        [keep_group[e // 2:e // 2 + 1] for e in range(E)], axis=0)

    neg = jnp.float32(-jnp.inf)
    masked = jnp.where(kg_expanded > 0, sc, neg)
    keep = rank_of(masked, E) < TOPK

    w = jnp.where(keep, s, 0.0)
    wsum = jnp.sum(w, axis=0, keepdims=True)
    wn = w / wsum * ROUTED_SCALING

    erow = jax.lax.broadcasted_iota(jnp.int32, (E, T), 0)
    e0 = jnp.min(jnp.where(keep, erow, E), axis=0, keepdims=True)
    e1 = jnp.max(jnp.where(keep, erow, -1), axis=0, keepdims=True)
    sel0 = erow == e0
    sel1 = erow == e1
    w0 = jnp.sum(jnp.where(sel0, wn, 0.0), axis=0, keepdims=True)
    w1 = jnp.sum(jnp.where(sel1, wn, 0.0), axis=0, keepdims=True)

    k_int = jnp.where(keep, 1, 0)
    csum = _lane_cumsum(k_int, T)
    excl = csum - k_int
    cnt = csum[:, T - 1:T]

    padded = ((cnt + (BTR - 1)) // BTR) * BTR
    incl = padded
    sh = 1
    while sh < E:
        incl = incl + jnp.concatenate(
            [jnp.zeros((sh, 1), jnp.int32), incl[:-sh]], axis=0)
        sh *= 2
    pstart_i = incl - padded
    run = incl[E - 1:E]

    d_e = pstart_i + excl
    dest0 = jnp.sum(jnp.where(sel0, d_e, 0), axis=0, keepdims=True)
    dest1 = jnp.sum(jnp.where(sel1, d_e, 0), axis=0, keepdims=True)
    dest_ref[0:1, :] = dest0
    dest_ref[1:2, :] = dest1

    bstart = jax.lax.broadcasted_iota(jnp.int32, (1, NBLK + 1), 1) * BTR
    be = jnp.full((1, NBLK + 1), -1, jnp.int32)
    for e in range(E):
        be = be + (pstart_i[e:e + 1] <= bstart).astype(jnp.int32)
    nreal = run // BTR
    lane = jax.lax.broadcasted_iota(jnp.int32, (1, NBLK + 1), 1)
    bexp_ref[...] = jnp.where(lane == NBLK, nreal, be)

    w01 = jnp.concatenate([w0, w1] + [jnp.zeros((1, T), jnp.float32)] * (E - 2),
                          axis=0)
    eye = jnp.eye(E, dtype=jnp.float32)
    wtok_ref[...] = lax.dot_general(w01, eye, (((0,), (0,)), ((), ())),
                                    preferred_element_type=jnp.float32)


@jax.jit
def _route(hidden_states, gate_w, bias_col):
    return pl.pallas_call(
        _route_body,
        grid=(1,),
        in_specs=[
            pl.BlockSpec((T, HIDDEN), lambda i: (0, 0)),
            pl.BlockSpec((HIDDEN, E), lambda i: (0, 0)),
            pl.BlockSpec((E, 1), lambda i: (0, 0)),
        ],
        out_specs=[
            pl.BlockSpec((T, E), lambda i: (0, 0)),
            pl.BlockSpec((2, T), lambda i: (0, 0)),
            pl.BlockSpec((1, NBLK + 1), lambda i: (0, 0)),
        ],
        out_shape=[
            jax.ShapeDtypeStruct((T, E), jnp.float32),
            jax.ShapeDtypeStruct((2, T), jnp.int32),
            jax.ShapeDtypeStruct((1, NBLK + 1), jnp.int32),
        ],
    )(hidden_states, gate_w, bias_col)


def _dispatch_body(hid_ref, dest_ref, sorted_ref, idx0, idx1, rows0, rows1,
                   rsem0, rsem1, wsem0, wsem1):
    wid = lax.axis_index("s") * SC_CORES + lax.axis_index("c")
    idx = (idx0, idx1)
    rows = (rows0, rows1)
    rsem = (rsem0, rsem1)
    wsem = (wsem0, wsem1)
    n = SC_CHUNKS_PER_WORKER

    def chunk_base(cc):
        base = (wid * n + cc) * SC_CHUNK
        return base, lax.rem(base, T)

    b0, t0 = chunk_base(0)
    pltpu.sync_copy(dest_ref.at[pl.ds(b0, SC_CHUNK)], idx[0])
    reads = [None, None]
    writes = [None, None]
    reads[0] = pltpu.async_copy(hid_ref.at[pl.ds(t0, SC_CHUNK)], rows[0],
                                rsem[0])
    for cc in range(n):
        cur = cc % 2
        nxt = 1 - cur
        if cc + 1 < n:
            if writes[nxt] is not None:
                writes[nxt].wait()
            bn, tn = chunk_base(cc + 1)
            pltpu.sync_copy(dest_ref.at[pl.ds(bn, SC_CHUNK)], idx[nxt])
            reads[nxt] = pltpu.async_copy(hid_ref.at[pl.ds(tn, SC_CHUNK)],
                                          rows[nxt], rsem[nxt])
        reads[cur].wait()
        writes[cur] = pltpu.async_copy(rows[cur], sorted_ref.at[idx[cur]],
                                       wsem[cur])
    for w in writes:
        if w is not None:
            w.wait()


@jax.jit
def _dispatch(hidden_states, dest_flat):
    mesh = plsc.VectorSubcoreMesh(core_axis_name="c", subcore_axis_name="s")
    fn = pl.kernel(
        _dispatch_body,
        out_type=jax.ShapeDtypeStruct((PBUF, HIDDEN), jnp.float32),
        mesh=mesh,
        scratch_types=[
            pltpu.VMEM((SC_CHUNK,), jnp.int32),
            pltpu.VMEM((SC_CHUNK,), jnp.int32),
            pltpu.VMEM((SC_CHUNK, HIDDEN), jnp.float32),
            pltpu.VMEM((SC_CHUNK, HIDDEN), jnp.float32),
            pltpu.SemaphoreType.DMA,
            pltpu.SemaphoreType.DMA,
            pltpu.SemaphoreType.DMA,
            pltpu.SemaphoreType.DMA,
        ],
    )
    return fn(hidden_states, dest_flat)


def _expert_body(bexp_ref, sorted_ref, wgu_ref, wd_ref, out_ref):
    @pl.when(pl.program_id(0) < bexp_ref[NBLK])
    def _():
        x = sorted_ref[...].astype(jnp.bfloat16)
        gu = jnp.dot(x, wgu_ref[0].astype(jnp.bfloat16),
                     preferred_element_type=jnp.float32)
        g = gu[:, :DFF]
        u = gu[:, DFF:]
        act = (g * jax.nn.sigmoid(g) * u).astype(jnp.bfloat16)
        out_ref[...] = jnp.dot(act, wd_ref[0].astype(jnp.bfloat16),
                               preferred_element_type=jnp.float32)


@jax.jit
def _expert_mlp(block_expert, sorted_h, w_gate_up, w_down):
    grid_spec = pltpu.PrefetchScalarGridSpec(
        num_scalar_prefetch=1,
        grid=(NBLK,),
        in_specs=[
            pl.BlockSpec((BTR, HIDDEN), lambda i, be: (i, 0)),
            pl.BlockSpec((1, HIDDEN, 2 * DFF), lambda i, be: (be[i], 0, 0)),
            pl.BlockSpec((1, DFF, HIDDEN), lambda i, be: (be[i], 0, 0)),
        ],
        out_specs=pl.BlockSpec((BTR, HIDDEN), lambda i, be: (i, 0)),
    )
    return pl.pallas_call(
        _expert_body,
        grid_spec=grid_spec,
        out_shape=jax.ShapeDtypeStruct((PBUF, HIDDEN), jnp.float32),
    )(block_expert, sorted_h, w_gate_up, w_down)


def _unpermute_body(rowout_ref, dest_ref, unsorted_ref, idx0, idx1, rows0,
                    rows1, rsem0, rsem1, wsem0, wsem1):
    wid = lax.axis_index("s") * SC_CORES + lax.axis_index("c")
    idx = (idx0, idx1)
    rows = (rows0, rows1)
    rsem = (rsem0, rsem1)
    wsem = (wsem0, wsem1)
    n = SC_CHUNKS_PER_WORKER

    b0 = wid * n * SC_CHUNK
    pltpu.sync_copy(dest_ref.at[pl.ds(b0, SC_CHUNK)], idx[0])
    reads = [None, None]
    writes = [None, None]
    reads[0] = pltpu.async_copy(rowout_ref.at[idx[0]], rows[0], rsem[0])
    for cc in range(n):
        cur = cc % 2
        nxt = 1 - cur
        base = (wid * n + cc) * SC_CHUNK
        if cc + 1 < n:
            if writes[nxt] is not None:
                writes[nxt].wait()
            bn = (wid * n + cc + 1) * SC_CHUNK
            pltpu.sync_copy(dest_ref.at[pl.ds(bn, SC_CHUNK)], idx[nxt])
            reads[nxt] = pltpu.async_copy(rowout_ref.at[idx[nxt]], rows[nxt],
                                          rsem[nxt])
        reads[cur].wait()
        writes[cur] = pltpu.async_copy(
            rows[cur], unsorted_ref.at[pl.ds(base, SC_CHUNK)], wsem[cur])
    for w in writes:
        if w is not None:
            w.wait()


@jax.jit
def _unpermute(rowout, dest_flat):
    mesh = plsc.VectorSubcoreMesh(core_axis_name="c", subcore_axis_name="s")
    fn = pl.kernel(
        _unpermute_body,
        out_type=jax.ShapeDtypeStruct((NPAIR, HIDDEN), jnp.float32),
        mesh=mesh,
        scratch_types=[
            pltpu.VMEM((SC_CHUNK,), jnp.int32),
            pltpu.VMEM((SC_CHUNK,), jnp.int32),
            pltpu.VMEM((SC_CHUNK, HIDDEN), jnp.float32),
            pltpu.VMEM((SC_CHUNK, HIDDEN), jnp.float32),
            pltpu.SemaphoreType.DMA,
            pltpu.SemaphoreType.DMA,
            pltpu.SemaphoreType.DMA,
            pltpu.SemaphoreType.DMA,
        ],
    )
    return fn(rowout, dest_flat)


def _shared_body(hid_ref, wshgu_ref, wshd_ref, out_ref):
    h = hid_ref[...].astype(jnp.bfloat16)
    gu = jnp.dot(h, wshgu_ref[...].astype(jnp.bfloat16),
                 preferred_element_type=jnp.float32)
    sg = gu[:, :2 * DFF]
    su = gu[:, 2 * DFF:]
    act = (sg * jax.nn.sigmoid(sg) * su).astype(jnp.bfloat16)
    out_ref[...] = jnp.dot(act, wshd_ref[...].astype(jnp.bfloat16),
                           preferred_element_type=jnp.float32)


def _make_shared_half(offset_blocks):
    nb = (T // 2) // BTS

    @jax.jit
    def fn(hidden_states, shared_w_gate_up, shared_w_down):
        return pl.pallas_call(
            _shared_body,
            grid=(nb,),
            in_specs=[
                pl.BlockSpec((BTS, HIDDEN),
                             lambda i, _o=offset_blocks: (i + _o, 0)),
                pl.BlockSpec((HIDDEN, 4 * DFF), lambda i: (0, 0)),
                pl.BlockSpec((2 * DFF, HIDDEN), lambda i: (0, 0)),
            ],
            out_specs=pl.BlockSpec((BTS, HIDDEN), lambda i: (i, 0)),
            out_shape=jax.ShapeDtypeStruct((T // 2, HIDDEN), jnp.float32),
        )(hidden_states, shared_w_gate_up, shared_w_down)

    return fn


_shared_mlp_a = _make_shared_half(0)
_shared_mlp_b = _make_shared_half((T // 2) // BTS)


def _combine_body(sha_ref, shb_ref, u0_ref, u1_ref, wtok_ref, out_ref):
    nba = (T // 2) // BTS
    w0 = wtok_ref[:, 0:1]
    w1 = wtok_ref[:, 1:2]
    sh = jnp.where(pl.program_id(0) < nba, sha_ref[...], shb_ref[...])
    out_ref[...] = sh + w0 * u0_ref[...] + w1 * u1_ref[...]


@jax.jit
def _combine(shared_a, shared_b, unsorted, wtok):
    nb = T // BTS
    nba = (T // 2) // BTS
    return pl.pallas_call(
        _combine_body,
        grid=(nb,),
        in_specs=[
            pl.BlockSpec((BTS, HIDDEN),
                         lambda i, _m=nba - 1: (jnp.minimum(i, _m), 0)),
            pl.BlockSpec((BTS, HIDDEN),
                         lambda i, _a=nba: (jnp.maximum(i - _a, 0), 0)),
            pl.BlockSpec((BTS, HIDDEN), lambda i: (i, 0)),
            pl.BlockSpec((BTS, HIDDEN), lambda i, _nb=nb: (i + _nb, 0)),
            pl.BlockSpec((BTS, E), lambda i: (i, 0)),
        ],
        out_specs=pl.BlockSpec((BTS, HIDDEN), lambda i: (i, 0)),
        out_shape=jax.ShapeDtypeStruct((T, HIDDEN), jnp.float32),
    )(shared_a, shared_b, unsorted, unsorted, wtok)


def kernel(hidden_states, gate_w, e_score_correction_bias, w_gate_up, w_down,
           shared_w_gate_up, shared_w_down):
    bias_col = e_score_correction_bias.reshape(E, 1)
    wtok, dest01, block_expert = _route(hidden_states, gate_w, bias_col)
    dest_flat = dest01.reshape(NPAIR)
    sorted_h = _dispatch(hidden_states, dest_flat)
    shared_a = _shared_mlp_a(hidden_states, shared_w_gate_up, shared_w_down)
    be, shared_a = lax.optimization_barrier(
        (block_expert.reshape(NBLK + 1), shared_a))
    rowout = _expert_mlp(be, sorted_h, w_gate_up, w_down)
    unsorted = _unpermute(rowout, dest_flat)
    shared_b = _shared_mlp_b(hidden_states, shared_w_gate_up, shared_w_down)
    return _combine(shared_a, shared_b, unsorted, wtok)
